# baseline (speedup 1.0000x reference)
import functools

import jax
import jax.numpy as jnp
from jax import lax
from jax.experimental import pallas as pl
from jax.experimental.pallas import tpu as pltpu
from jax.experimental.pallas import tpu_sc as plsc

VOCAB = 100000
EMB = 64
BATCH = 1024
CTX = 50

_NC = 2
_NS = 16
_NW = _NC * _NS
_RPW = BATCH // _NW
_INV_CTX = 1.0 / CTX


def _pool_body(ctx_hbm, table_hbm, out_hbm, idx_v, rows_v, pooled_v, sem):
    wid = lax.axis_index("s") * _NC + lax.axis_index("c")
    base = wid * _RPW
    pltpu.sync_copy(ctx_hbm.at[pl.ds(base, _RPW)], idx_v)

    def row(i, _):
        pltpu.async_copy(table_hbm.at[idx_v.at[i]], rows_v, sem).wait()

        def acc_step(j, carry):
            a0, a1, a2, a3 = carry
            return (a0 + rows_v[j, pl.ds(0, 16)],
                    a1 + rows_v[j, pl.ds(16, 16)],
                    a2 + rows_v[j, pl.ds(32, 16)],
                    a3 + rows_v[j, pl.ds(48, 16)])

        z = jnp.zeros((16,), jnp.float32)
        a0, a1, a2, a3 = lax.fori_loop(0, CTX, acc_step, (z, z, z, z))
        s = jnp.float32(_INV_CTX)
        pooled_v[i, pl.ds(0, 16)] = a0 * s
        pooled_v[i, pl.ds(16, 16)] = a1 * s
        pooled_v[i, pl.ds(32, 16)] = a2 * s
        pooled_v[i, pl.ds(48, 16)] = a3 * s
        return 0

    lax.fori_loop(0, _RPW, row, 0)
    pltpu.sync_copy(pooled_v, out_hbm.at[pl.ds(base, _RPW)])


_pool = functools.partial(
    pl.kernel,
    out_type=jax.ShapeDtypeStruct((BATCH, EMB), jnp.float32),
    mesh=plsc.VectorSubcoreMesh(core_axis_name="c", subcore_axis_name="s"),
    scratch_types=[
        pltpu.VMEM((_RPW, CTX), jnp.int32),
        pltpu.VMEM((CTX, EMB), jnp.float32),
        pltpu.VMEM((_RPW, EMB), jnp.float32),
        pltpu.SemaphoreType.DMA,
    ],
    compiler_params=pltpu.CompilerParams(use_tc_tiling_on_sc=False),
)(_pool_body)


_VB = 2048


def _mm_body(p_ref, w_ref, b_ref, o_ref):
    o_ref[...] = (
        jnp.dot(p_ref[...], w_ref[...], preferred_element_type=jnp.float32)
        + b_ref[...]
    )


def _project(pooled, W, b2d):
    grid = pl.cdiv(VOCAB, _VB)
    return pl.pallas_call(
        _mm_body,
        grid=(grid,),
        in_specs=[
            pl.BlockSpec((BATCH, EMB), lambda i: (0, 0)),
            pl.BlockSpec((EMB, _VB), lambda i: (0, i)),
            pl.BlockSpec((1, _VB), lambda i: (0, i)),
        ],
        out_specs=pl.BlockSpec((BATCH, _VB), lambda i: (0, i)),
        out_shape=jax.ShapeDtypeStruct((BATCH, VOCAB), jnp.float32),
    )(pooled, W, b2d)


def kernel(context, emb_table, W, b):
    ctx = context.astype(jnp.int32)
    pooled = _pool(ctx, emb_table)
    return _project(pooled, W, b.reshape(1, VOCAB))


# speedup vs baseline: 1.0003x; 1.0003x over previous
import functools

import jax
import jax.numpy as jnp
from jax import lax
from jax.experimental import pallas as pl
from jax.experimental.pallas import tpu as pltpu
from jax.experimental.pallas import tpu_sc as plsc

VOCAB = 100000
EMB = 64
BATCH = 1024
CTX = 50

_NC = 2
_NS = 16
_NW = _NC * _NS
_RPW = BATCH // _NW
_INV_CTX = 1.0 / CTX


def _pool_body(ctx_hbm, table_hbm, out_hbm, idx_v, rows_v, pooled_v, sem):
    wid = lax.axis_index("s") * _NC + lax.axis_index("c")
    base = wid * _RPW
    pltpu.sync_copy(ctx_hbm.at[pl.ds(base, _RPW)], idx_v)

    def row(i, _):
        pltpu.async_copy(table_hbm.at[idx_v.at[i]], rows_v, sem).wait()

        def acc_step(j, carry):
            a0, a1, a2, a3 = carry
            return (a0 + rows_v[j, pl.ds(0, 16)],
                    a1 + rows_v[j, pl.ds(16, 16)],
                    a2 + rows_v[j, pl.ds(32, 16)],
                    a3 + rows_v[j, pl.ds(48, 16)])

        z = jnp.zeros((16,), jnp.float32)
        a0, a1, a2, a3 = lax.fori_loop(0, CTX, acc_step, (z, z, z, z))
        s = jnp.float32(_INV_CTX)
        pooled_v[i, pl.ds(0, 16)] = a0 * s
        pooled_v[i, pl.ds(16, 16)] = a1 * s
        pooled_v[i, pl.ds(32, 16)] = a2 * s
        pooled_v[i, pl.ds(48, 16)] = a3 * s
        return 0

    lax.fori_loop(0, _RPW, row, 0)
    pltpu.sync_copy(pooled_v, out_hbm.at[pl.ds(base, _RPW)])


_pool = functools.partial(
    pl.kernel,
    out_type=jax.ShapeDtypeStruct((BATCH, EMB), jnp.float32),
    mesh=plsc.VectorSubcoreMesh(core_axis_name="c", subcore_axis_name="s"),
    scratch_types=[
        pltpu.VMEM((_RPW, CTX), jnp.int32),
        pltpu.VMEM((CTX, EMB), jnp.float32),
        pltpu.VMEM((_RPW, EMB), jnp.float32),
        pltpu.SemaphoreType.DMA,
    ],
    compiler_params=pltpu.CompilerParams(use_tc_tiling_on_sc=False),
)(_pool_body)


_VB = 4096


def _mm_body(p_ref, w_ref, b_ref, o_ref):
    o_ref[...] = (
        jnp.dot(p_ref[...], w_ref[...], preferred_element_type=jnp.float32)
        + b_ref[...]
    )


def _project(pooled, W, b2d):
    grid = pl.cdiv(VOCAB, _VB)
    return pl.pallas_call(
        _mm_body,
        grid=(grid,),
        in_specs=[
            pl.BlockSpec((BATCH, EMB), lambda i: (0, 0)),
            pl.BlockSpec((EMB, _VB), lambda i: (0, i)),
            pl.BlockSpec((1, _VB), lambda i: (0, i)),
        ],
        out_specs=pl.BlockSpec((BATCH, _VB), lambda i: (0, i)),
        out_shape=jax.ShapeDtypeStruct((BATCH, VOCAB), jnp.float32),
    )(pooled, W, b2d)


def kernel(context, emb_table, W, b):
    ctx = context.astype(jnp.int32)
    pooled = _pool(ctx, emb_table)
    return _project(pooled, W, b.reshape(1, VOCAB))


# speedup vs baseline: 1.0011x; 1.0008x over previous
import functools

import jax
import jax.numpy as jnp
from jax import lax
from jax.experimental import pallas as pl
from jax.experimental.pallas import tpu as pltpu
from jax.experimental.pallas import tpu_sc as plsc

VOCAB = 100000
EMB = 64
BATCH = 1024
CTX = 50

_NC = 2
_NS = 16
_NW = _NC * _NS
_RPW = BATCH // _NW
_INV_CTX = 1.0 / CTX


def _pool_body(ctx_hbm, table_hbm, out_hbm, idx_v, rows_v, pooled_v, sem):
    wid = lax.axis_index("s") * _NC + lax.axis_index("c")
    base = wid * _RPW
    pltpu.sync_copy(ctx_hbm.at[pl.ds(base, _RPW)], idx_v)

    def row(i, _):
        pltpu.async_copy(table_hbm.at[idx_v.at[i]], rows_v, sem).wait()

        def acc_step(j, carry):
            a0, a1, a2, a3 = carry
            return (a0 + rows_v[j, pl.ds(0, 16)],
                    a1 + rows_v[j, pl.ds(16, 16)],
                    a2 + rows_v[j, pl.ds(32, 16)],
                    a3 + rows_v[j, pl.ds(48, 16)])

        z = jnp.zeros((16,), jnp.float32)
        a0, a1, a2, a3 = lax.fori_loop(0, CTX, acc_step, (z, z, z, z))
        s = jnp.float32(_INV_CTX)
        pooled_v[i, pl.ds(0, 16)] = a0 * s
        pooled_v[i, pl.ds(16, 16)] = a1 * s
        pooled_v[i, pl.ds(32, 16)] = a2 * s
        pooled_v[i, pl.ds(48, 16)] = a3 * s
        return 0

    lax.fori_loop(0, _RPW, row, 0)
    pltpu.sync_copy(pooled_v, out_hbm.at[pl.ds(base, _RPW)])


_pool = functools.partial(
    pl.kernel,
    out_type=jax.ShapeDtypeStruct((BATCH, EMB), jnp.float32),
    mesh=plsc.VectorSubcoreMesh(core_axis_name="c", subcore_axis_name="s"),
    scratch_types=[
        pltpu.VMEM((_RPW, CTX), jnp.int32),
        pltpu.VMEM((CTX, EMB), jnp.float32),
        pltpu.VMEM((_RPW, EMB), jnp.float32),
        pltpu.SemaphoreType.DMA,
    ],
    compiler_params=pltpu.CompilerParams(use_tc_tiling_on_sc=False),
)(_pool_body)


_BM = 32
_NSTEPS = BATCH // _BM
_NBUF = 2


def _mm_body(p_ref, w_ref, b_ref, o_hbm, acc, sem):
    i = pl.program_id(0)
    slot = jax.lax.rem(i, _NBUF)

    @pl.when(i >= _NBUF)
    def _drain_slot():
        j = i - _NBUF
        pltpu.make_async_copy(
            acc.at[slot], o_hbm.at[pl.ds(j * _BM, _BM)], sem.at[slot]
        ).wait()

    acc[slot] = (
        jnp.dot(p_ref[...], w_ref[...], preferred_element_type=jnp.float32)
        + b_ref[...]
    )

    pltpu.make_async_copy(
        acc.at[slot], o_hbm.at[pl.ds(i * _BM, _BM)], sem.at[slot]
    ).start()

    @pl.when(i == _NSTEPS - 1)
    def _drain_all():
        for j in range(_NSTEPS - _NBUF, _NSTEPS):
            s = j % _NBUF
            pltpu.make_async_copy(
                acc.at[s], o_hbm.at[pl.ds(j * _BM, _BM)], sem.at[s]
            ).wait()


def _project(pooled, W, b2d):
    return pl.pallas_call(
        _mm_body,
        grid=(_NSTEPS,),
        in_specs=[
            pl.BlockSpec((_BM, EMB), lambda i: (i, 0)),
            pl.BlockSpec((EMB, VOCAB), lambda i: (0, 0)),
            pl.BlockSpec((1, VOCAB), lambda i: (0, 0)),
        ],
        out_specs=pl.BlockSpec(memory_space=pl.ANY),
        out_shape=jax.ShapeDtypeStruct((BATCH, VOCAB), jnp.float32),
        scratch_shapes=[
            pltpu.VMEM((_NBUF, _BM, VOCAB), jnp.float32),
            pltpu.SemaphoreType.DMA((_NBUF,)),
        ],
        compiler_params=pltpu.CompilerParams(vmem_limit_bytes=110 * 1024 * 1024),
    )(pooled, W, b2d)


def kernel(context, emb_table, W, b):
    ctx = context.astype(jnp.int32)
    pooled = _pool(ctx, emb_table)
    return _project(pooled, W, b.reshape(1, VOCAB))


# speedup vs baseline: 2.1604x; 2.1580x over previous
import functools

import jax
import jax.numpy as jnp
from jax import lax
from jax.experimental import pallas as pl
from jax.experimental.pallas import tpu as pltpu
from jax.experimental.pallas import tpu_sc as plsc

VOCAB = 100000
EMB = 64
BATCH = 1024
CTX = 50

_NC = 2
_NS = 16
_NW = _NC * _NS
_RPW = BATCH // _NW
_INV_CTX = 1.0 / CTX


def _pool_body(ctx_hbm, table_hbm, out_hbm, idx_v, rows_v, pooled_v, sem):
    wid = lax.axis_index("s") * _NC + lax.axis_index("c")
    base = wid * _RPW
    pltpu.sync_copy(ctx_hbm.at[pl.ds(base, _RPW)], idx_v)

    def row(i, _):
        pltpu.async_copy(table_hbm.at[idx_v.at[i]], rows_v, sem).wait()

        def acc_step(j, carry):
            a0, a1, a2, a3 = carry
            return (a0 + rows_v[j, pl.ds(0, 16)],
                    a1 + rows_v[j, pl.ds(16, 16)],
                    a2 + rows_v[j, pl.ds(32, 16)],
                    a3 + rows_v[j, pl.ds(48, 16)])

        z = jnp.zeros((16,), jnp.float32)
        a0, a1, a2, a3 = lax.fori_loop(0, CTX, acc_step, (z, z, z, z))
        s = jnp.float32(_INV_CTX)
        pooled_v[i, pl.ds(0, 16)] = a0 * s
        pooled_v[i, pl.ds(16, 16)] = a1 * s
        pooled_v[i, pl.ds(32, 16)] = a2 * s
        pooled_v[i, pl.ds(48, 16)] = a3 * s
        return 0

    lax.fori_loop(0, _RPW, row, 0)
    pltpu.sync_copy(pooled_v, out_hbm.at[pl.ds(base, _RPW)])


_pool = functools.partial(
    pl.kernel,
    out_type=jax.ShapeDtypeStruct((BATCH, EMB), jnp.float32),
    mesh=plsc.VectorSubcoreMesh(core_axis_name="c", subcore_axis_name="s"),
    scratch_types=[
        pltpu.VMEM((_RPW, CTX), jnp.int32),
        pltpu.VMEM((CTX, EMB), jnp.float32),
        pltpu.VMEM((_RPW, EMB), jnp.float32),
        pltpu.SemaphoreType.DMA,
    ],
    compiler_params=pltpu.CompilerParams(use_tc_tiling_on_sc=False),
)(_pool_body)


_VB = 2048
_NSTEPS = pl.cdiv(VOCAB, _VB)


def _mm_body(p_ref, w_ref, b_ref, ot_ref):
    ot_ref[...] = (
        jax.lax.dot_general(
            w_ref[...], p_ref[...],
            dimension_numbers=(((0,), (1,)), ((), ())),
            preferred_element_type=jnp.float32,
        )
        + b_ref[...]
    )


def _project(pooled, W, bcol):
    return pl.pallas_call(
        _mm_body,
        grid=(_NSTEPS,),
        in_specs=[
            pl.BlockSpec((BATCH, EMB), lambda i: (0, 0)),
            pl.BlockSpec((EMB, _VB), lambda i: (0, i)),
            pl.BlockSpec((_VB, 1), lambda i: (i, 0)),
        ],
        out_specs=pl.BlockSpec((_VB, BATCH), lambda i: (i, 0)),
        out_shape=jax.ShapeDtypeStruct((VOCAB, BATCH), jnp.float32),
        compiler_params=pltpu.CompilerParams(vmem_limit_bytes=110 * 1024 * 1024),
    )(pooled, W, bcol)


def kernel(context, emb_table, W, b):
    ctx = context.astype(jnp.int32)
    pooled = _pool(ctx, emb_table)
    return _project(pooled, W, b.reshape(VOCAB, 1)).T


# speedup vs baseline: 2.4344x; 1.1268x over previous
import functools

import jax
import jax.numpy as jnp
from jax import lax
from jax.experimental import pallas as pl
from jax.experimental.pallas import tpu as pltpu
from jax.experimental.pallas import tpu_sc as plsc

VOCAB = 100000
EMB = 64
BATCH = 1024
CTX = 50

_NC = 2
_NS = 16
_NW = _NC * _NS
_RPW = BATCH // _NW
_INV_CTX = 1.0 / CTX


def _pool_body(ctx_hbm, table_hbm, out_hbm, idx_v, rows_v, pooled_v, sem):
    wid = lax.axis_index("s") * _NC + lax.axis_index("c")
    base = wid * _RPW
    pltpu.sync_copy(ctx_hbm.at[pl.ds(base, _RPW)], idx_v)

    def row(i, _):
        pltpu.async_copy(table_hbm.at[idx_v.at[i]], rows_v, sem).wait()

        def acc_step(j, carry):
            a0, a1, a2, a3 = carry
            return (a0 + rows_v[j, pl.ds(0, 16)],
                    a1 + rows_v[j, pl.ds(16, 16)],
                    a2 + rows_v[j, pl.ds(32, 16)],
                    a3 + rows_v[j, pl.ds(48, 16)])

        z = jnp.zeros((16,), jnp.float32)
        a0, a1, a2, a3 = lax.fori_loop(0, CTX, acc_step, (z, z, z, z))
        s = jnp.float32(_INV_CTX)
        pooled_v[i, pl.ds(0, 16)] = a0 * s
        pooled_v[i, pl.ds(16, 16)] = a1 * s
        pooled_v[i, pl.ds(32, 16)] = a2 * s
        pooled_v[i, pl.ds(48, 16)] = a3 * s
        return 0

    lax.fori_loop(0, _RPW, row, 0)
    pltpu.sync_copy(pooled_v, out_hbm.at[pl.ds(base, _RPW)])


_pool = functools.partial(
    pl.kernel,
    out_type=jax.ShapeDtypeStruct((BATCH, EMB), jnp.float32),
    mesh=plsc.VectorSubcoreMesh(core_axis_name="c", subcore_axis_name="s"),
    scratch_types=[
        pltpu.VMEM((_RPW, CTX), jnp.int32),
        pltpu.VMEM((CTX, EMB), jnp.float32),
        pltpu.VMEM((_RPW, EMB), jnp.float32),
        pltpu.SemaphoreType.DMA,
    ],
    compiler_params=pltpu.CompilerParams(use_tc_tiling_on_sc=False),
)(_pool_body)


_VB = 2048
_NSTEPS = pl.cdiv(VOCAB, _VB)


def _mm_body(p_ref, w_ref, b_ref, ot_ref):
    w_aug = jnp.concatenate([w_ref[...], b_ref[...]], axis=0)
    ones = jnp.ones((BATCH, 1), jnp.float32)
    p_aug = jnp.concatenate([p_ref[...], ones], axis=1)
    ot_ref[...] = jax.lax.dot_general(
        w_aug, p_aug,
        dimension_numbers=(((0,), (1,)), ((), ())),
        preferred_element_type=jnp.float32,
    )


def _project(pooled, W, brow):
    return pl.pallas_call(
        _mm_body,
        grid=(_NSTEPS,),
        in_specs=[
            pl.BlockSpec((BATCH, EMB), lambda i: (0, 0)),
            pl.BlockSpec((EMB, _VB), lambda i: (0, i)),
            pl.BlockSpec((1, _VB), lambda i: (0, i)),
        ],
        out_specs=pl.BlockSpec((_VB, BATCH), lambda i: (i, 0)),
        out_shape=jax.ShapeDtypeStruct((VOCAB, BATCH), jnp.float32),
        compiler_params=pltpu.CompilerParams(vmem_limit_bytes=110 * 1024 * 1024),
    )(pooled, W, brow)


def kernel(context, emb_table, W, b):
    ctx = context.astype(jnp.int32)
    pooled = _pool(ctx, emb_table)
    return _project(pooled, W, b.reshape(1, VOCAB)).T


# speedup vs baseline: 2.6283x; 1.0797x over previous
import functools

import jax
import jax.numpy as jnp
from jax import lax
from jax.experimental import pallas as pl
from jax.experimental.pallas import tpu as pltpu
from jax.experimental.pallas import tpu_sc as plsc

VOCAB = 100000
EMB = 64
BATCH = 1024
CTX = 50

_NC = 2
_NS = 16
_NW = _NC * _NS
_RPW = BATCH // _NW
_INV_CTX = 1.0 / CTX


def _pool_body(ctx_hbm, table_hbm, out_hbm, idx_v, rows_v, pooled_v, sem):
    wid = lax.axis_index("s") * _NC + lax.axis_index("c")
    base = wid * _RPW
    pltpu.sync_copy(ctx_hbm.at[pl.ds(base, _RPW)], idx_v)

    def fire(i, _):
        pltpu.make_async_copy(
            table_hbm.at[idx_v.at[i]], rows_v.at[i], sem
        ).start()
        return 0

    def drain(i, _):
        pltpu.make_async_copy(
            table_hbm.at[idx_v.at[i]], rows_v.at[i], sem
        ).wait()
        return 0

    _CH = 8
    lax.fori_loop(0, _CH, fire, 0)
    for c in range(_RPW // _CH):
        if (c + 1) * _CH < _RPW:
            lax.fori_loop((c + 1) * _CH, (c + 2) * _CH, fire, 0)
        lax.fori_loop(c * _CH, (c + 1) * _CH, drain, 0)

    def row(i, _):
        def acc_step(j, carry):
            a0, a1, a2, a3 = carry
            return (a0 + rows_v[i, j, pl.ds(0, 16)],
                    a1 + rows_v[i, j, pl.ds(16, 16)],
                    a2 + rows_v[i, j, pl.ds(32, 16)],
                    a3 + rows_v[i, j, pl.ds(48, 16)])

        z = jnp.zeros((16,), jnp.float32)
        a0, a1, a2, a3 = lax.fori_loop(0, CTX, acc_step, (z, z, z, z))
        s = jnp.float32(_INV_CTX)
        pooled_v[i, pl.ds(0, 16)] = a0 * s
        pooled_v[i, pl.ds(16, 16)] = a1 * s
        pooled_v[i, pl.ds(32, 16)] = a2 * s
        pooled_v[i, pl.ds(48, 16)] = a3 * s
        return 0

    lax.fori_loop(0, _RPW, row, 0)
    pltpu.sync_copy(pooled_v, out_hbm.at[pl.ds(base, _RPW)])


_pool = functools.partial(
    pl.kernel,
    out_type=jax.ShapeDtypeStruct((BATCH, EMB), jnp.float32),
    mesh=plsc.VectorSubcoreMesh(core_axis_name="c", subcore_axis_name="s"),
    scratch_types=[
        pltpu.VMEM((_RPW, CTX), jnp.int32),
        pltpu.VMEM((_RPW, CTX, EMB), jnp.float32),
        pltpu.VMEM((_RPW, EMB), jnp.float32),
        pltpu.SemaphoreType.DMA,
    ],
    compiler_params=pltpu.CompilerParams(use_tc_tiling_on_sc=False),
)(_pool_body)


_VB = 2048
_NSTEPS = pl.cdiv(VOCAB, _VB)


def _mm_body(p_ref, w_ref, b_ref, ot_ref):
    w_aug = jnp.concatenate([w_ref[...], b_ref[...]], axis=0)
    ones = jnp.ones((BATCH, 1), jnp.float32)
    p_aug = jnp.concatenate([p_ref[...], ones], axis=1)
    ot_ref[...] = jax.lax.dot_general(
        w_aug, p_aug,
        dimension_numbers=(((0,), (1,)), ((), ())),
        preferred_element_type=jnp.float32,
    )


def _project(pooled, W, brow):
    return pl.pallas_call(
        _mm_body,
        grid=(_NSTEPS,),
        in_specs=[
            pl.BlockSpec((BATCH, EMB), lambda i: (0, 0)),
            pl.BlockSpec((EMB, _VB), lambda i: (0, i)),
            pl.BlockSpec((1, _VB), lambda i: (0, i)),
        ],
        out_specs=pl.BlockSpec((_VB, BATCH), lambda i: (i, 0)),
        out_shape=jax.ShapeDtypeStruct((VOCAB, BATCH), jnp.float32),
        compiler_params=pltpu.CompilerParams(vmem_limit_bytes=110 * 1024 * 1024),
    )(pooled, W, brow)


def kernel(context, emb_table, W, b):
    ctx = context.astype(jnp.int32)
    pooled = _pool(ctx, emb_table)
    return _project(pooled, W, b.reshape(1, VOCAB)).T


# speedup vs baseline: 2.7357x; 1.0409x over previous
import functools

import jax
import jax.numpy as jnp
from jax import lax
from jax.experimental import pallas as pl
from jax.experimental.pallas import tpu as pltpu
from jax.experimental.pallas import tpu_sc as plsc

VOCAB = 100000
EMB = 64
BATCH = 1024
CTX = 50

_NC = 2
_NS = 16
_NW = _NC * _NS
_RPW = BATCH // _NW
_INV_CTX = 1.0 / CTX
_CH = 8
_NCHUNK = _RPW // _CH


def _pool_body(ctx_hbm, table_hbm, out_hbm, idx_v, rows_v, pooled_v, sem):
    wid = lax.axis_index("s") * _NC + lax.axis_index("c")
    base = wid * _RPW
    pltpu.sync_copy(ctx_hbm.at[pl.ds(base, _RPW)], idx_v)

    def fire(i, _):
        buf = lax.rem(lax.div(i, _CH), 2)
        pltpu.make_async_copy(
            table_hbm.at[idx_v.at[i]], rows_v.at[buf, lax.rem(i, _CH)], sem
        ).start()
        return 0

    def drain(i, _):
        buf = lax.rem(lax.div(i, _CH), 2)
        pltpu.make_async_copy(
            table_hbm.at[idx_v.at[i]], rows_v.at[buf, lax.rem(i, _CH)], sem
        ).wait()
        return 0

    def row(i, _):
        buf = lax.rem(lax.div(i, _CH), 2)
        r = lax.rem(i, _CH)

        def acc_step(j, carry):
            a0, a1, a2, a3 = carry
            return (a0 + rows_v[buf, r, j, pl.ds(0, 16)],
                    a1 + rows_v[buf, r, j, pl.ds(16, 16)],
                    a2 + rows_v[buf, r, j, pl.ds(32, 16)],
                    a3 + rows_v[buf, r, j, pl.ds(48, 16)])

        z = jnp.zeros((16,), jnp.float32)
        a0, a1, a2, a3 = lax.fori_loop(0, CTX, acc_step, (z, z, z, z))
        s = jnp.float32(_INV_CTX)
        pooled_v[i, pl.ds(0, 16)] = a0 * s
        pooled_v[i, pl.ds(16, 16)] = a1 * s
        pooled_v[i, pl.ds(32, 16)] = a2 * s
        pooled_v[i, pl.ds(48, 16)] = a3 * s
        return 0

    lax.fori_loop(0, _CH, fire, 0)
    for c in range(_NCHUNK):
        if c + 1 < _NCHUNK:
            lax.fori_loop((c + 1) * _CH, (c + 2) * _CH, fire, 0)
        lax.fori_loop(c * _CH, (c + 1) * _CH, drain, 0)
        lax.fori_loop(c * _CH, (c + 1) * _CH, row, 0)

    pltpu.sync_copy(pooled_v, out_hbm.at[pl.ds(base, _RPW)])


_pool = functools.partial(
    pl.kernel,
    out_type=jax.ShapeDtypeStruct((BATCH, EMB), jnp.float32),
    mesh=plsc.VectorSubcoreMesh(core_axis_name="c", subcore_axis_name="s"),
    scratch_types=[
        pltpu.VMEM((_RPW, CTX), jnp.int32),
        pltpu.VMEM((2, _CH, CTX, 128), jnp.float32),
        pltpu.VMEM((_RPW, EMB), jnp.float32),
        pltpu.SemaphoreType.DMA,
    ],
    compiler_params=pltpu.CompilerParams(use_tc_tiling_on_sc=True),
)(_pool_body)


_VB = 2048
_NSTEPS = pl.cdiv(VOCAB, _VB)


def _mm_body(p_ref, w_ref, b_ref, ot_ref):
    w_aug = jnp.concatenate([w_ref[...], b_ref[...]], axis=0)
    ones = jnp.ones((BATCH, 1), jnp.float32)
    p_aug = jnp.concatenate([p_ref[...], ones], axis=1)
    ot_ref[...] = jax.lax.dot_general(
        w_aug, p_aug,
        dimension_numbers=(((0,), (1,)), ((), ())),
        preferred_element_type=jnp.float32,
    )


def _project(pooled, W, brow):
    return pl.pallas_call(
        _mm_body,
        grid=(_NSTEPS,),
        in_specs=[
            pl.BlockSpec((BATCH, EMB), lambda i: (0, 0)),
            pl.BlockSpec((EMB, _VB), lambda i: (0, i)),
            pl.BlockSpec((1, _VB), lambda i: (0, i)),
        ],
        out_specs=pl.BlockSpec((_VB, BATCH), lambda i: (i, 0)),
        out_shape=jax.ShapeDtypeStruct((VOCAB, BATCH), jnp.float32),
        compiler_params=pltpu.CompilerParams(vmem_limit_bytes=110 * 1024 * 1024),
    )(pooled, W, brow)


def kernel(context, emb_table, W, b):
    ctx = context.astype(jnp.int32)
    table_pad = jnp.pad(emb_table, ((0, 0), (0, 128 - EMB)))
    pooled = _pool(ctx, table_pad)
    return _project(pooled, W, b.reshape(1, VOCAB)).T


# speedup vs baseline: 2.9619x; 1.0827x over previous
import functools

import jax
import jax.numpy as jnp
from jax import lax
from jax.experimental import pallas as pl
from jax.experimental.pallas import tpu as pltpu
from jax.experimental.pallas import tpu_sc as plsc

VOCAB = 100000
EMB = 64
BATCH = 1024
CTX = 50

_NC = 2
_NS = 16
_NW = _NC * _NS
_RPW = BATCH // _NW
_INV_CTX = 1.0 / CTX
_CH = 8
_NCHUNK = _RPW // _CH


def _pool_body(ctx_hbm, table_hbm, out_hbm, idx_v, rows_v, pooled_v, sem):
    wid = lax.axis_index("s") * _NC + lax.axis_index("c")
    base = wid * _RPW
    pltpu.sync_copy(ctx_hbm.at[pl.ds(base, _RPW)], idx_v)

    def fire(i, _):
        buf = lax.rem(lax.div(i, _CH), 2)
        pltpu.make_async_copy(
            table_hbm.at[idx_v.at[i]], rows_v.at[buf, lax.rem(i, _CH)], sem
        ).start()
        return 0

    def drain(i, _):
        buf = lax.rem(lax.div(i, _CH), 2)
        pltpu.make_async_copy(
            table_hbm.at[idx_v.at[i]], rows_v.at[buf, lax.rem(i, _CH)], sem
        ).wait()
        return 0

    def row(i, _):
        buf = lax.rem(lax.div(i, _CH), 2)
        r = lax.rem(i, _CH)

        def acc_step(j, carry):
            a0, a1, a2, a3 = carry
            return (a0 + rows_v[buf, r, j, pl.ds(0, 16)],
                    a1 + rows_v[buf, r, j, pl.ds(16, 16)],
                    a2 + rows_v[buf, r, j, pl.ds(32, 16)],
                    a3 + rows_v[buf, r, j, pl.ds(48, 16)])

        z = jnp.zeros((16,), jnp.float32)
        a0, a1, a2, a3 = lax.fori_loop(0, CTX, acc_step, (z, z, z, z))
        s = jnp.float32(_INV_CTX)
        pooled_v[i, pl.ds(0, 16)] = a0 * s
        pooled_v[i, pl.ds(16, 16)] = a1 * s
        pooled_v[i, pl.ds(32, 16)] = a2 * s
        pooled_v[i, pl.ds(48, 16)] = a3 * s
        return 0

    lax.fori_loop(0, _CH, fire, 0)
    for c in range(_NCHUNK):
        if c + 1 < _NCHUNK:
            lax.fori_loop((c + 1) * _CH, (c + 2) * _CH, fire, 0)
        lax.fori_loop(c * _CH, (c + 1) * _CH, drain, 0)
        lax.fori_loop(c * _CH, (c + 1) * _CH, row, 0)

    pltpu.sync_copy(pooled_v, out_hbm.at[pl.ds(base, _RPW)])


_pool = functools.partial(
    pl.kernel,
    out_type=jax.ShapeDtypeStruct((BATCH, EMB), jnp.float32),
    mesh=plsc.VectorSubcoreMesh(core_axis_name="c", subcore_axis_name="s"),
    scratch_types=[
        pltpu.VMEM((_RPW, CTX), jnp.int32),
        pltpu.VMEM((2, _CH, CTX, 128), jnp.float32),
        pltpu.VMEM((_RPW, EMB), jnp.float32),
        pltpu.SemaphoreType.DMA,
    ],
    compiler_params=pltpu.CompilerParams(use_tc_tiling_on_sc=True),
)(_pool_body)


_RB = 4096
_RSTEPS = pl.cdiv(VOCAB, _RB)


def _repack_body(et_ref, tp_ref):
    eye = jnp.float32(1.0) * (
        jax.lax.broadcasted_iota(jnp.int32, (EMB, EMB), 0)
        == jax.lax.broadcasted_iota(jnp.int32, (EMB, EMB), 1)
    )
    rows = jax.lax.dot_general(
        et_ref[...], eye,
        dimension_numbers=(((0,), (0,)), ((), ())),
        preferred_element_type=jnp.float32,
    )
    tp_ref[:, 0:EMB] = rows


def _repack(embT):
    return pl.pallas_call(
        _repack_body,
        grid=(_RSTEPS,),
        in_specs=[pl.BlockSpec((EMB, _RB), lambda i: (0, i))],
        out_specs=pl.BlockSpec((_RB, 128), lambda i: (i, 0)),
        out_shape=jax.ShapeDtypeStruct((VOCAB, 128), jnp.float32),
    )(embT)


_VB = 2048
_NSTEPS = pl.cdiv(VOCAB, _VB)


def _mm_body(p_ref, w_ref, b_ref, ot_ref):
    w_aug = jnp.concatenate([w_ref[...], b_ref[...]], axis=0)
    ones = jnp.ones((BATCH, 1), jnp.float32)
    p_aug = jnp.concatenate([p_ref[...], ones], axis=1)
    ot_ref[...] = jax.lax.dot_general(
        w_aug, p_aug,
        dimension_numbers=(((0,), (1,)), ((), ())),
        preferred_element_type=jnp.float32,
    )


def _project(pooled, W, brow):
    return pl.pallas_call(
        _mm_body,
        grid=(_NSTEPS,),
        in_specs=[
            pl.BlockSpec((BATCH, EMB), lambda i: (0, 0)),
            pl.BlockSpec((EMB, _VB), lambda i: (0, i)),
            pl.BlockSpec((1, _VB), lambda i: (0, i)),
        ],
        out_specs=pl.BlockSpec((_VB, BATCH), lambda i: (i, 0)),
        out_shape=jax.ShapeDtypeStruct((VOCAB, BATCH), jnp.float32),
        compiler_params=pltpu.CompilerParams(vmem_limit_bytes=110 * 1024 * 1024),
    )(pooled, W, brow)


def kernel(context, emb_table, W, b):
    ctx = context.astype(jnp.int32)
    table_pad = _repack(emb_table.T)
    pooled = _pool(ctx, table_pad)
    return _project(pooled, W, b.reshape(1, VOCAB)).T
